# Initial kernel scaffold; baseline (speedup 1.0000x reference)
import jax
import jax.numpy as jnp
from jax.experimental import pallas as pl


def kernel(input, table):
    raise NotImplementedError("write your pallas kernel here")


# speedup vs baseline: 1.5098x; 1.5098x over previous
import jax
import jax.numpy as jnp
from jax import lax
from jax.experimental import pallas as pl
from jax.experimental.pallas import tpu as pltpu
from jax.experimental.pallas import tpu_sc as plsc

EMBED_DIM = 32
ROW_BYTES = EMBED_DIM * 4
CHUNK = 128
NUM_CORES = 2
NUM_SUBCORES = 16
NUM_WORKERS = NUM_CORES * NUM_SUBCORES


def kernel(input, table):
    batch, hist = input.shape
    num_indices = batch * hist
    b_per_w = num_indices // NUM_WORKERS
    assert b_per_w % (2 * CHUNK) == 0
    indices = input.reshape(num_indices)
    table_pad = jnp.pad(table, ((0, 0), (0, 128 - EMBED_DIM)))

    mesh = plsc.VectorSubcoreMesh(core_axis_name="core",
                                  subcore_axis_name="subcore")

    @pl.kernel(
        out_type=jax.ShapeDtypeStruct((num_indices, 128), jnp.float32),
        mesh=mesh,
        scratch_types=[
            pltpu.VMEM((b_per_w,), jnp.int32),
            pltpu.VMEM((CHUNK, 128), jnp.float32),
            pltpu.VMEM((CHUNK, 128), jnp.float32),
            pltpu.SemaphoreType.DMA,
            pltpu.SemaphoreType.DMA,
        ],
    )
    def gather_kernel(tab_hbm, idx_hbm, out_hbm, idx_v, rows0, rows1,
                      sem0, sem1):
        wid = lax.axis_index("subcore") * NUM_CORES + lax.axis_index("core")
        base = wid * b_per_w
        pltpu.sync_copy(idx_hbm.at[pl.ds(base, b_per_w)], idx_v)

        @pl.loop(0, b_per_w, step=2 * CHUNK)
        def _(c):
            g0 = pltpu.async_copy(
                tab_hbm.at[idx_v.at[pl.ds(c, CHUNK)]], rows0, sem0)
            g1 = pltpu.async_copy(
                tab_hbm.at[idx_v.at[pl.ds(c + CHUNK, CHUNK)]], rows1, sem1)
            g0.wait()
            pltpu.sync_copy(rows0, out_hbm.at[pl.ds(base + c, CHUNK)])
            g1.wait()
            pltpu.sync_copy(rows1, out_hbm.at[pl.ds(base + c + CHUNK, CHUNK)])

    out_pad = gather_kernel(table_pad, indices)
    return out_pad[:, :EMBED_DIM].reshape(batch, hist, EMBED_DIM)
